# baseline (speedup 1.0000x reference)
import functools

import jax
import jax.numpy as jnp
from jax import lax
from jax.experimental import pallas as pl
from jax.experimental.pallas import tpu as pltpu
from jax.experimental.pallas import tpu_sc as plsc

_B = 16384
_D = 16
_PACK = 128 // _D
_CHUNK = 128
_L = 16


@functools.lru_cache(maxsize=None)
def _build(NC: int, NS: int):
    NW = NC * NS
    b_per_w = _B // NW
    n_chunks = b_per_w // _CHUNK
    lines_per_w = b_per_w * _D // 128
    mesh = plsc.VectorSubcoreMesh(core_axis_name="c", subcore_axis_name="s")

    @functools.partial(
        pl.kernel,
        mesh=mesh,
        compiler_params=pltpu.CompilerParams(needs_layout_passes=False),
        out_type=(
            jax.ShapeDtypeStruct((_B * _D // 128, 128), jnp.float32),
            jax.ShapeDtypeStruct((_B * _D // 128, 128), jnp.float32),
        ),
        scratch_types=[
            pltpu.VMEM((b_per_w,), jnp.int32),
            pltpu.VMEM((b_per_w,), jnp.int32),
            pltpu.VMEM((b_per_w,), jnp.int32),
            pltpu.VMEM((b_per_w,), jnp.int32),
            pltpu.VMEM((_CHUNK, 128), jnp.float32),
            pltpu.VMEM((_CHUNK, 128), jnp.float32),
            pltpu.VMEM((_CHUNK, 128), jnp.float32),
            pltpu.VMEM((_CHUNK, 128), jnp.float32),
            pltpu.VMEM((lines_per_w, 128), jnp.float32),
            pltpu.VMEM((lines_per_w, 128), jnp.float32),
            pltpu.SemaphoreType.DMA,
            pltpu.SemaphoreType.DMA,
            pltpu.SemaphoreType.DMA,
            pltpu.SemaphoreType.DMA,
        ],
    )
    def k(idx_hbm, pidx_hbm, W_hbm, Wp_hbm, out_hbm, pout_hbm,
          idx_v, pidx_v, qidx_v, pqidx_v, lines0_v, lines1_v, plines0_v,
          plines1_v, out_v, pout_v, sem_a0, sem_a1, sem_b0, sem_b1):
        wid = lax.axis_index("s") * NC + lax.axis_index("c")
        base = wid * b_per_w
        pltpu.sync_copy(idx_hbm.at[pl.ds(base, b_per_w)], idx_v)
        pltpu.sync_copy(pidx_hbm.at[pl.ds(base, b_per_w)], pidx_v)

        for g in range(b_per_w // _L):
            s = pl.ds(g * _L, _L)
            qidx_v[s] = lax.shift_right_logical(idx_v[s], 3)
            pqidx_v[s] = lax.shift_right_logical(pidx_v[s], 3)

        sems_a = (sem_a0, sem_a1)
        sems_b = (sem_b0, sem_b1)
        lines = (lines0_v, lines1_v)
        plines = (plines0_v, plines1_v)

        def fire(c):
            s = pl.ds(c * _CHUNK, _CHUNK)
            return (
                pltpu.async_copy(W_hbm.at[qidx_v.at[s]], lines[c % 2], sems_a[c % 2]),
                pltpu.async_copy(Wp_hbm.at[pqidx_v.at[s]], plines[c % 2], sems_b[c % 2]),
            )

        def extract(c, bufs, sidx_v, dst_v):
            buf = bufs[c % 2]
            for g in range(_CHUNK // _L):
                vidx = sidx_v[pl.ds(c * _CHUNK + g * _L, _L)]
                base_col = lax.shift_left(jnp.bitwise_and(vidx, _PACK - 1), 4)
                elem = lax.iota(jnp.int32, _L) + (c * _CHUNK + g * _L)
                rows = lax.iota(jnp.int32, _L) + g * _L
                orow = lax.shift_right_logical(elem, 3)
                ocol0 = lax.shift_left(jnp.bitwise_and(elem, _PACK - 1), 4)
                for d in range(_D):
                    vals = plsc.load_gather(buf, [rows, base_col + d])
                    plsc.store_scatter(dst_v, [orow, ocol0 + d], vals)

        pending = fire(0)
        for c in range(n_chunks):
            nxt = fire(c + 1) if c + 1 < n_chunks else None
            for cp in pending:
                cp.wait()
            extract(c, lines, idx_v, out_v)
            extract(c, plines, pidx_v, pout_v)
            pending = nxt

        pltpu.sync_copy(out_v, out_hbm.at[pl.ds(wid * lines_per_w, lines_per_w)])
        pltpu.sync_copy(pout_v, pout_hbm.at[pl.ds(wid * lines_per_w, lines_per_w)])

    return k


def kernel(batch, persona_batch, W, W_persona):
    info = plsc.get_sparse_core_info()
    NC, NS = info.num_cores, info.num_subcores
    out, pout = _build(NC, NS)(
        batch.astype(jnp.int32),
        persona_batch.astype(jnp.int32),
        W.reshape(-1, 128),
        W_persona.reshape(-1, 128),
    )
    return out.reshape(_B, _D), pout.reshape(_B, _D)
